# baseline (speedup 1.0000x reference)
import math

import jax
import jax.numpy as jnp
from jax.experimental import pallas as pl
from jax.experimental.pallas import tpu as pltpu

N_TOKENS = 32768
IN_FEATURES = 2048
OUT_FEATURES = 2048
RANK = 16
NUM_EXPERTS = 64
TOP_K = 8
SCALING = 32 / 16

BLK = 1024
SUB = 256
_INV_SQRT2 = 1.0 / math.sqrt(2.0)
_NEG_KEY = -2147483648


def _fused_body(x_ref, raT_ref, evT_ref, bwT_ref, out_ref):
    x = x_ref[...]
    sxa = jnp.dot(x, raT_ref[...], preferred_element_type=jnp.float32)
    t = sxa.T
    s = t[:NUM_EXPERTS, :]
    xa = t[NUM_EXPERTS:NUM_EXPERTS + RANK, :]

    wnum_parts = []
    denom_parts = []
    for c in range(BLK // SUB):
        sc = s[:, c * SUB:(c + 1) * SUB]
        row = jax.lax.broadcasted_iota(jnp.int32, sc.shape, 0)
        u = jax.lax.bitcast_convert_type(sc, jnp.int32)
        key = u ^ ((u >> 31) & jnp.int32(0x7FFFFFFF))
        cur = (key & jnp.int32(~0x3F)) | (jnp.int32(NUM_EXPERTS - 1) - row)

        m1 = jnp.max(sc, axis=0, keepdims=True)
        ex = jnp.exp(sc - m1)

        for j in range(TOP_K):
            mkey = jnp.max(cur, axis=0, keepdims=True)
            cur = jnp.where(cur == mkey, jnp.int32(_NEG_KEY), cur)

        wnum_parts.append(jnp.where(cur == jnp.int32(_NEG_KEY), ex, 0.0))
        denom_parts.append(jnp.sum(wnum_parts[-1], axis=0, keepdims=True))

    wnum = jnp.concatenate(wnum_parts, axis=1)
    denom = jnp.concatenate(denom_parts, axis=1)

    etok = jnp.dot(evT_ref[...], wnum, preferred_element_type=jnp.float32)
    h = xa + etok / denom
    g = 0.5 * h * (1.0 + jax.lax.erf(h * _INV_SQRT2))
    out_ref[...] = jax.lax.dot_general(
        g, bwT_ref[...], (((0,), (0,)), ((), ())),
        preferred_element_type=jnp.float32)


def kernel(x, A_w, B_w, expert_vectors, router_w):
    n = x.shape[0]
    grid = n // BLK
    raT = jnp.zeros((IN_FEATURES, 128), jnp.float32)
    raT = raT.at[:, :NUM_EXPERTS].set(router_w.T)
    raT = raT.at[:, NUM_EXPERTS:NUM_EXPERTS + RANK].set(A_w.T)
    evT = expert_vectors.T
    bwT = B_w.T * SCALING
    return pl.pallas_call(
        _fused_body,
        grid=(grid,),
        in_specs=[
            pl.BlockSpec((BLK, IN_FEATURES), lambda i: (i, 0)),
            pl.BlockSpec((IN_FEATURES, 128), lambda i: (0, 0)),
            pl.BlockSpec((RANK, NUM_EXPERTS), lambda i: (0, 0)),
            pl.BlockSpec((RANK, OUT_FEATURES), lambda i: (0, 0)),
        ],
        out_specs=pl.BlockSpec((BLK, OUT_FEATURES), lambda i: (i, 0)),
        compiler_params=pltpu.CompilerParams(dimension_semantics=("parallel",)),
        out_shape=jax.ShapeDtypeStruct((n, OUT_FEATURES), jnp.float32),
    )(x, raT, evT, bwT)
